# baseline (speedup 1.0000x reference)
import functools

import jax
import jax.numpy as jnp
from jax import lax
from jax.experimental import pallas as pl
from jax.experimental.pallas import tpu as pltpu
from jax.experimental.pallas import tpu_sc as plsc

T, D, E, K, FF, FFS, C = 2048, 1024, 64, 2, 704, 2816, 128
NC, NS, L = 2, 16, 16
NW = NC * NS
BT = 256
NB = T // BT
TPW = T // NW
RB = 32
CH = 16
XROWS = (E + 1) * C


def _sigmoid(v):
    return 1.0 / (1.0 + jnp.exp(-v))


def _router_rank_body(x_ref, gw_ref, tril_ref, sd1_ref, sd2_ref,
                      sc1_ref, sc2_ref, w1_ref, w2_ref, carry_ref):
    b = pl.program_id(0)

    @pl.when(b == 0)
    def _():
        carry_ref[...] = jnp.zeros_like(carry_ref)

    x = x_ref[...]
    logits = jnp.dot(x, gw_ref[...], preferred_element_type=jnp.float32)
    m = jnp.max(logits, axis=-1, keepdims=True)
    ex = jnp.exp(logits - m)
    probs = ex / jnp.sum(ex, axis=-1, keepdims=True)

    iota = lax.broadcasted_iota(jnp.int32, probs.shape, 1)
    m1 = jnp.max(probs, axis=-1, keepdims=True)
    i1 = jnp.min(jnp.where(probs == m1, iota, E), axis=-1, keepdims=True)
    probs2 = jnp.where(iota == i1, -jnp.inf, probs)
    m2 = jnp.max(probs2, axis=-1, keepdims=True)
    i2 = jnp.min(jnp.where(probs2 == m2, iota, E), axis=-1, keepdims=True)

    oh1 = (i1 == iota).astype(jnp.float32)
    oh2 = (i2 == iota).astype(jnp.float32)
    s = oh1 + oh2
    csum_ex = (jnp.dot(tril_ref[...], s, preferred_element_type=jnp.float32)
               + carry_ref[0:1, :])
    carry_ref[0:1, :] = carry_ref[0:1, :] + jnp.sum(s, axis=0, keepdims=True)

    r1 = jnp.sum(oh1 * csum_ex, axis=-1, keepdims=True).astype(jnp.int32)
    r2 = jnp.sum(oh2 * csum_ex, axis=-1, keepdims=True).astype(jnp.int32)

    tt = lax.broadcasted_iota(jnp.int32, r1.shape, 0)
    sink = E * C + (b * BT + tt) // TPW
    k1 = r1 < C
    k2 = r2 < C
    slot1 = i1 * C + r1
    slot2 = i2 * C + r2
    sd1_ref[0, 0, :] = jnp.where(k1, slot1, sink)[:, 0]
    sd2_ref[0, 0, :] = jnp.where(k2, slot2, sink)[:, 0]
    sc1_ref[0, 0, :] = jnp.where(k1, slot1, 0)[:, 0]
    sc2_ref[0, 0, :] = jnp.where(k2, slot2, 0)[:, 0]
    w1_ref[0, :, :] = jnp.broadcast_to(jnp.where(k1, m1, 0.0), (BT, L))
    w2_ref[0, :, :] = jnp.broadcast_to(jnp.where(k2, m2, 0.0), (BT, L))


def _router_rank(x, gate_w, tril):
    outs = pl.pallas_call(
        _router_rank_body,
        grid=(NB,),
        in_specs=[pl.BlockSpec((BT, D), lambda b: (b, 0)),
                  pl.BlockSpec((D, E), lambda b: (0, 0)),
                  pl.BlockSpec((BT, BT), lambda b: (0, 0))],
        out_specs=[pl.BlockSpec((1, 1, BT), lambda b: (b, 0, 0))] * 4
        + [pl.BlockSpec((1, BT, L), lambda b: (b, 0, 0))] * 2,
        out_shape=[jax.ShapeDtypeStruct((NB, 1, BT), jnp.int32),
                   jax.ShapeDtypeStruct((NB, 1, BT), jnp.int32),
                   jax.ShapeDtypeStruct((NB, 1, BT), jnp.int32),
                   jax.ShapeDtypeStruct((NB, 1, BT), jnp.int32),
                   jax.ShapeDtypeStruct((NB, BT, L), jnp.float32),
                   jax.ShapeDtypeStruct((NB, BT, L), jnp.float32)],
        scratch_shapes=[pltpu.VMEM((8, E), jnp.float32)],
    )(x, gate_w, tril)
    return (outs[0].reshape(T), outs[1].reshape(T), outs[2].reshape(T),
            outs[3].reshape(T), outs[4].reshape(T, L), outs[5].reshape(T, L))


def _dispatch(sd1, sd2, x):
    mesh = plsc.VectorSubcoreMesh(core_axis_name="c", subcore_axis_name="s")

    @functools.partial(
        pl.kernel,
        mesh=mesh,
        out_type=jax.ShapeDtypeStruct((XROWS, D), jnp.float32),
        scratch_types=(
            pltpu.VMEM((RB, D), jnp.float32),
            pltpu.VMEM((RB,), jnp.int32),
            pltpu.VMEM((RB,), jnp.int32),
            pltpu.SemaphoreType.DMA,
        ),
    )
    def k(sd1_hbm, sd2_hbm, x_hbm, xe_hbm, rows_v, i1_v, i2_v, sem):
        wid = lax.axis_index("s") * NC + lax.axis_index("c")
        for ch in range(TPW // RB):
            base = wid * TPW + ch * RB
            pltpu.sync_copy(x_hbm.at[pl.ds(base, RB)], rows_v)
            pltpu.sync_copy(sd1_hbm.at[pl.ds(base, RB)], i1_v)
            pltpu.sync_copy(sd2_hbm.at[pl.ds(base, RB)], i2_v)
            pltpu.async_copy(rows_v, xe_hbm.at[i1_v], sem).wait()
            pltpu.async_copy(rows_v, xe_hbm.at[i2_v], sem).wait()

    return k(sd1, sd2, x)


D2 = D // 2


def _experts_body(x_ref, w1a_ref, w1b_ref, w3a_ref, w3b_ref,
                  w2a_ref, w2b_ref, o_ref):
    x = x_ref[...]
    xl = x[:, :D2]
    xr = x[:, D2:]
    g = (jnp.dot(xl, w1a_ref[0], preferred_element_type=jnp.float32)
         + jnp.dot(xr, w1b_ref[0], preferred_element_type=jnp.float32))
    u = (jnp.dot(xl, w3a_ref[0], preferred_element_type=jnp.float32)
         + jnp.dot(xr, w3b_ref[0], preferred_element_type=jnp.float32))
    h = g * _sigmoid(g) * u
    o_ref[:, :D2] = jnp.dot(h, w2a_ref[0], preferred_element_type=jnp.float32)
    o_ref[:, D2:] = jnp.dot(h, w2b_ref[0], preferred_element_type=jnp.float32)


def _experts(xe, w1, w3, w2):
    return pl.pallas_call(
        _experts_body,
        grid=(E,),
        in_specs=[pl.BlockSpec((C, D), lambda e: (e, 0)),
                  pl.BlockSpec((1, D2, FF), lambda e: (e, 0, 0)),
                  pl.BlockSpec((1, D2, FF), lambda e: (e, 1, 0)),
                  pl.BlockSpec((1, D2, FF), lambda e: (e, 0, 0)),
                  pl.BlockSpec((1, D2, FF), lambda e: (e, 1, 0)),
                  pl.BlockSpec((1, FF, D2), lambda e: (e, 0, 0)),
                  pl.BlockSpec((1, FF, D2), lambda e: (e, 0, 1))],
        out_specs=pl.BlockSpec((C, D), lambda e: (e, 0)),
        out_shape=jax.ShapeDtypeStruct((E * C, D), jnp.float32),
        compiler_params=pltpu.CompilerParams(
            dimension_semantics=("parallel",)),
    )(xe, w1, w1, w3, w3, w2, w2)


def _shared_body(x_ref, w1_ref, w3_ref, w2_ref, gw_ref, o_ref):
    x = x_ref[...]
    s1 = jnp.dot(x, w1_ref[...], preferred_element_type=jnp.float32)
    s3 = jnp.dot(x, w3_ref[...], preferred_element_type=jnp.float32)
    h = s1 * _sigmoid(s1) * s3
    so = jnp.dot(h, w2_ref[...], preferred_element_type=jnp.float32)
    gl = jnp.dot(x, gw_ref[...], preferred_element_type=jnp.float32)
    o_ref[...] = so * _sigmoid(gl)


def _shared(x, sw1, sw3, sw2, sgate_w):
    return pl.pallas_call(
        _shared_body,
        grid=(T // BT,),
        in_specs=[pl.BlockSpec((BT, D), lambda i: (i, 0)),
                  pl.BlockSpec((D, FFS), lambda i: (0, 0)),
                  pl.BlockSpec((D, FFS), lambda i: (0, 0)),
                  pl.BlockSpec((FFS, D), lambda i: (0, 0)),
                  pl.BlockSpec((D, 1), lambda i: (0, 0))],
        out_specs=pl.BlockSpec((BT, D), lambda i: (i, 0)),
        out_shape=jax.ShapeDtypeStruct((T, D), jnp.float32),
        compiler_params=pltpu.CompilerParams(
            dimension_semantics=("parallel",)),
    )(x, sw1, sw3, sw2, sgate_w)


def _combine(oe, sc1, sc2, w1p, w2p, so):
    mesh = plsc.VectorSubcoreMesh(core_axis_name="c", subcore_axis_name="s")

    @functools.partial(
        pl.kernel,
        mesh=mesh,
        out_type=jax.ShapeDtypeStruct((T, D), jnp.float32),
        scratch_types=(
            pltpu.VMEM((CH,), jnp.int32),
            pltpu.VMEM((CH,), jnp.int32),
            pltpu.VMEM((CH, L), jnp.float32),
            pltpu.VMEM((CH, L), jnp.float32),
            pltpu.VMEM((CH, D), jnp.float32),
            pltpu.VMEM((CH, D), jnp.float32),
            pltpu.VMEM((CH, D), jnp.float32),
            pltpu.VMEM((CH, D), jnp.float32),
            pltpu.SemaphoreType.DMA,
        ),
    )
    def k(oe_hbm, sc1_hbm, sc2_hbm, w1_hbm, w2_hbm, so_hbm, out_hbm,
          s1_v, s2_v, w1_v, w2_v, ra_v, rb_v, so_v, o_v, sem):
        wid = lax.axis_index("s") * NC + lax.axis_index("c")
        for ch in range(TPW // CH):
            tb = wid * TPW + ch * CH
            pltpu.sync_copy(sc1_hbm.at[pl.ds(tb, CH)], s1_v)
            pltpu.sync_copy(sc2_hbm.at[pl.ds(tb, CH)], s2_v)
            pltpu.sync_copy(w1_hbm.at[pl.ds(tb, CH)], w1_v)
            pltpu.sync_copy(w2_hbm.at[pl.ds(tb, CH)], w2_v)
            pltpu.async_copy(oe_hbm.at[s1_v], ra_v, sem).wait()
            pltpu.async_copy(oe_hbm.at[s2_v], rb_v, sem).wait()
            pltpu.sync_copy(so_hbm.at[pl.ds(tb, CH)], so_v)

            def tok_body(t, _):
                wa = w1_v[t, pl.ds(0, L)]
                wb = w2_v[t, pl.ds(0, L)]
                ma = wa != 0.0
                mb = wb != 0.0
                for j in range(D // L):
                    a = ra_v[t, pl.ds(j * L, L)]
                    bb = rb_v[t, pl.ds(j * L, L)]
                    s = so_v[t, pl.ds(j * L, L)]
                    o_v[t, pl.ds(j * L, L)] = (
                        jnp.where(ma, a * wa, 0.0)
                        + jnp.where(mb, bb * wb, 0.0) + s)
                return 0

            lax.fori_loop(0, CH, tok_body, 0)
            pltpu.sync_copy(o_v, out_hbm.at[pl.ds(tb, CH)])

    return k(oe, sc1, sc2, w1p, w2p, so)


def kernel(hidden_states, gate_w, w1, w3, w2, sw1, sw3, sw2, sgate_w):
    tril = jnp.tril(jnp.ones((BT, BT), jnp.float32), -1)
    sd1, sd2, sc1, sc2, w1p, w2p = _router_rank(hidden_states, gate_w, tril)
    xe = _dispatch(sd1, sd2, hidden_states)
    oe = _experts(xe, w1, w3, w2)
    so = _shared(hidden_states, sw1, sw3, sw2, sgate_w)
    return _combine(oe, sc1, sc2, w1p, w2p, so)


# speedup vs baseline: 1.0064x; 1.0064x over previous
import functools

import jax
import jax.numpy as jnp
from jax import lax
from jax.experimental import pallas as pl
from jax.experimental.pallas import tpu as pltpu
from jax.experimental.pallas import tpu_sc as plsc

T, D, E, K, FF, FFS, C = 2048, 1024, 64, 2, 704, 2816, 128
NC, NS, L = 2, 16, 16
NW = NC * NS
BT = 256
NB = T // BT
TPW = T // NW
RB = 32
CH = 16
XROWS = (E + 1) * C


def _sigmoid(v):
    return 1.0 / (1.0 + jnp.exp(-v))


def _router_rank_body(x_ref, gw_ref, tril_ref, sd1_ref, sd2_ref,
                      sc1_ref, sc2_ref, w1_ref, w2_ref, carry_ref):
    b = pl.program_id(0)

    @pl.when(b == 0)
    def _():
        carry_ref[...] = jnp.zeros_like(carry_ref)

    x = x_ref[...]
    logits = jnp.dot(x, gw_ref[...], preferred_element_type=jnp.float32)
    m = jnp.max(logits, axis=-1, keepdims=True)
    ex = jnp.exp(logits - m)
    probs = ex / jnp.sum(ex, axis=-1, keepdims=True)

    iota = lax.broadcasted_iota(jnp.int32, probs.shape, 1)
    m1 = jnp.max(probs, axis=-1, keepdims=True)
    i1 = jnp.min(jnp.where(probs == m1, iota, E), axis=-1, keepdims=True)
    probs2 = jnp.where(iota == i1, -jnp.inf, probs)
    m2 = jnp.max(probs2, axis=-1, keepdims=True)
    i2 = jnp.min(jnp.where(probs2 == m2, iota, E), axis=-1, keepdims=True)

    oh1 = (i1 == iota).astype(jnp.float32)
    oh2 = (i2 == iota).astype(jnp.float32)
    s = oh1 + oh2
    csum_ex = (jnp.dot(tril_ref[...], s, preferred_element_type=jnp.float32)
               + carry_ref[0:1, :])
    carry_ref[0:1, :] = carry_ref[0:1, :] + jnp.sum(s, axis=0, keepdims=True)

    r1 = jnp.sum(oh1 * csum_ex, axis=-1, keepdims=True).astype(jnp.int32)
    r2 = jnp.sum(oh2 * csum_ex, axis=-1, keepdims=True).astype(jnp.int32)

    tt = lax.broadcasted_iota(jnp.int32, r1.shape, 0)
    sink = E * C + (b * BT + tt) // TPW
    k1 = r1 < C
    k2 = r2 < C
    slot1 = i1 * C + r1
    slot2 = i2 * C + r2
    sd1_ref[0, 0, :] = jnp.where(k1, slot1, sink)[:, 0]
    sd2_ref[0, 0, :] = jnp.where(k2, slot2, sink)[:, 0]
    sc1_ref[0, 0, :] = jnp.where(k1, slot1, 0)[:, 0]
    sc2_ref[0, 0, :] = jnp.where(k2, slot2, 0)[:, 0]
    w1_ref[0, :, :] = jnp.broadcast_to(jnp.where(k1, m1, 0.0), (BT, L))
    w2_ref[0, :, :] = jnp.broadcast_to(jnp.where(k2, m2, 0.0), (BT, L))


def _router_rank(x, gate_w, tril):
    outs = pl.pallas_call(
        _router_rank_body,
        grid=(NB,),
        in_specs=[pl.BlockSpec((BT, D), lambda b: (b, 0)),
                  pl.BlockSpec((D, E), lambda b: (0, 0)),
                  pl.BlockSpec((BT, BT), lambda b: (0, 0))],
        out_specs=[pl.BlockSpec((1, 1, BT), lambda b: (b, 0, 0))] * 4
        + [pl.BlockSpec((1, BT, L), lambda b: (b, 0, 0))] * 2,
        out_shape=[jax.ShapeDtypeStruct((NB, 1, BT), jnp.int32),
                   jax.ShapeDtypeStruct((NB, 1, BT), jnp.int32),
                   jax.ShapeDtypeStruct((NB, 1, BT), jnp.int32),
                   jax.ShapeDtypeStruct((NB, 1, BT), jnp.int32),
                   jax.ShapeDtypeStruct((NB, BT, L), jnp.float32),
                   jax.ShapeDtypeStruct((NB, BT, L), jnp.float32)],
        scratch_shapes=[pltpu.VMEM((8, E), jnp.float32)],
    )(x, gate_w, tril)
    return (outs[0].reshape(T), outs[1].reshape(T), outs[2].reshape(T),
            outs[3].reshape(T), outs[4].reshape(T, L), outs[5].reshape(T, L))


def _dispatch(sd1, sd2, x):
    mesh = plsc.VectorSubcoreMesh(core_axis_name="c", subcore_axis_name="s")

    @functools.partial(
        pl.kernel,
        mesh=mesh,
        out_type=jax.ShapeDtypeStruct((XROWS, D), jnp.float32),
        scratch_types=(
            pltpu.VMEM((RB, D), jnp.float32),
            pltpu.VMEM((RB,), jnp.int32),
            pltpu.VMEM((RB,), jnp.int32),
            pltpu.SemaphoreType.DMA,
        ),
    )
    def k(sd1_hbm, sd2_hbm, x_hbm, xe_hbm, rows_v, i1_v, i2_v, sem):
        wid = lax.axis_index("s") * NC + lax.axis_index("c")
        for ch in range(TPW // RB):
            base = wid * TPW + ch * RB
            pltpu.sync_copy(x_hbm.at[pl.ds(base, RB)], rows_v)
            pltpu.sync_copy(sd1_hbm.at[pl.ds(base, RB)], i1_v)
            pltpu.sync_copy(sd2_hbm.at[pl.ds(base, RB)], i2_v)
            c1 = pltpu.async_copy(rows_v, xe_hbm.at[i1_v], sem)
            c2 = pltpu.async_copy(rows_v, xe_hbm.at[i2_v], sem)
            c1.wait()
            c2.wait()

    return k(sd1, sd2, x)


D2 = D // 2


def _experts_body(x_ref, w1a_ref, w1b_ref, w3a_ref, w3b_ref,
                  w2a_ref, w2b_ref, o_ref):
    x = x_ref[...]
    xl = x[:, :D2]
    xr = x[:, D2:]
    g = (jnp.dot(xl, w1a_ref[0], preferred_element_type=jnp.float32)
         + jnp.dot(xr, w1b_ref[0], preferred_element_type=jnp.float32))
    u = (jnp.dot(xl, w3a_ref[0], preferred_element_type=jnp.float32)
         + jnp.dot(xr, w3b_ref[0], preferred_element_type=jnp.float32))
    h = g * _sigmoid(g) * u
    o_ref[:, :D2] = jnp.dot(h, w2a_ref[0], preferred_element_type=jnp.float32)
    o_ref[:, D2:] = jnp.dot(h, w2b_ref[0], preferred_element_type=jnp.float32)


def _experts(xe, w1, w3, w2):
    return pl.pallas_call(
        _experts_body,
        grid=(E,),
        in_specs=[pl.BlockSpec((C, D), lambda e: (e, 0)),
                  pl.BlockSpec((1, D2, FF), lambda e: (e, 0, 0)),
                  pl.BlockSpec((1, D2, FF), lambda e: (e, 1, 0)),
                  pl.BlockSpec((1, D2, FF), lambda e: (e, 0, 0)),
                  pl.BlockSpec((1, D2, FF), lambda e: (e, 1, 0)),
                  pl.BlockSpec((1, FF, D2), lambda e: (e, 0, 0)),
                  pl.BlockSpec((1, FF, D2), lambda e: (e, 0, 1))],
        out_specs=pl.BlockSpec((C, D), lambda e: (e, 0)),
        out_shape=jax.ShapeDtypeStruct((E * C, D), jnp.float32),
        compiler_params=pltpu.CompilerParams(
            dimension_semantics=("parallel",)),
    )(xe, w1, w1, w3, w3, w2, w2)


def _shared_body(x_ref, w1_ref, w3_ref, w2_ref, gw_ref, o_ref):
    x = x_ref[...]
    s1 = jnp.dot(x, w1_ref[...], preferred_element_type=jnp.float32)
    s3 = jnp.dot(x, w3_ref[...], preferred_element_type=jnp.float32)
    h = s1 * _sigmoid(s1) * s3
    so = jnp.dot(h, w2_ref[...], preferred_element_type=jnp.float32)
    gl = jnp.dot(x, gw_ref[...], preferred_element_type=jnp.float32)
    o_ref[...] = so * _sigmoid(gl)


def _shared(x, sw1, sw3, sw2, sgate_w):
    return pl.pallas_call(
        _shared_body,
        grid=(T // BT,),
        in_specs=[pl.BlockSpec((BT, D), lambda i: (i, 0)),
                  pl.BlockSpec((D, FFS), lambda i: (0, 0)),
                  pl.BlockSpec((D, FFS), lambda i: (0, 0)),
                  pl.BlockSpec((FFS, D), lambda i: (0, 0)),
                  pl.BlockSpec((D, 1), lambda i: (0, 0))],
        out_specs=pl.BlockSpec((BT, D), lambda i: (i, 0)),
        out_shape=jax.ShapeDtypeStruct((T, D), jnp.float32),
        compiler_params=pltpu.CompilerParams(
            dimension_semantics=("parallel",)),
    )(x, sw1, sw3, sw2, sgate_w)


def _combine(oe, sc1, sc2, w1p, w2p, so):
    mesh = plsc.VectorSubcoreMesh(core_axis_name="c", subcore_axis_name="s")

    @functools.partial(
        pl.kernel,
        mesh=mesh,
        out_type=jax.ShapeDtypeStruct((T, D), jnp.float32),
        scratch_types=(
            pltpu.VMEM((CH,), jnp.int32),
            pltpu.VMEM((CH,), jnp.int32),
            pltpu.VMEM((CH, L), jnp.float32),
            pltpu.VMEM((CH, L), jnp.float32),
            pltpu.VMEM((CH, D), jnp.float32),
            pltpu.VMEM((CH, D), jnp.float32),
            pltpu.VMEM((CH, D), jnp.float32),
            pltpu.VMEM((CH, D), jnp.float32),
            pltpu.SemaphoreType.DMA,
        ),
    )
    def k(oe_hbm, sc1_hbm, sc2_hbm, w1_hbm, w2_hbm, so_hbm, out_hbm,
          s1_v, s2_v, w1_v, w2_v, ra_v, rb_v, so_v, o_v, sem):
        wid = lax.axis_index("s") * NC + lax.axis_index("c")
        for ch in range(TPW // CH):
            tb = wid * TPW + ch * CH
            pltpu.sync_copy(sc1_hbm.at[pl.ds(tb, CH)], s1_v)
            pltpu.sync_copy(sc2_hbm.at[pl.ds(tb, CH)], s2_v)
            pltpu.sync_copy(w1_hbm.at[pl.ds(tb, CH)], w1_v)
            pltpu.sync_copy(w2_hbm.at[pl.ds(tb, CH)], w2_v)
            g1 = pltpu.async_copy(oe_hbm.at[s1_v], ra_v, sem)
            g2 = pltpu.async_copy(oe_hbm.at[s2_v], rb_v, sem)
            pltpu.sync_copy(so_hbm.at[pl.ds(tb, CH)], so_v)
            g1.wait()
            g2.wait()

            def tok_body(t, _):
                wa = w1_v[t, pl.ds(0, L)]
                wb = w2_v[t, pl.ds(0, L)]
                ma = wa != 0.0
                mb = wb != 0.0
                for j in range(D // L):
                    a = ra_v[t, pl.ds(j * L, L)]
                    bb = rb_v[t, pl.ds(j * L, L)]
                    s = so_v[t, pl.ds(j * L, L)]
                    o_v[t, pl.ds(j * L, L)] = (
                        jnp.where(ma, a * wa, 0.0)
                        + jnp.where(mb, bb * wb, 0.0) + s)
                return 0

            lax.fori_loop(0, CH, tok_body, 0)
            pltpu.sync_copy(o_v, out_hbm.at[pl.ds(tb, CH)])

    return k(oe, sc1, sc2, w1p, w2p, so)


def kernel(hidden_states, gate_w, w1, w3, w2, sw1, sw3, sw2, sgate_w):
    tril = jnp.tril(jnp.ones((BT, BT), jnp.float32), -1)
    sd1, sd2, sc1, sc2, w1p, w2p = _router_rank(hidden_states, gate_w, tril)
    xe = _dispatch(sd1, sd2, hidden_states)
    oe = _experts(xe, w1, w3, w2)
    so = _shared(hidden_states, sw1, sw3, sw2, sgate_w)
    return _combine(oe, sc1, sc2, w1p, w2p, so)


# speedup vs baseline: 1.0292x; 1.0227x over previous
import functools

import jax
import jax.numpy as jnp
from jax import lax
from jax.experimental import pallas as pl
from jax.experimental.pallas import tpu as pltpu
from jax.experimental.pallas import tpu_sc as plsc

T, D, E, K, FF, FFS, C = 2048, 1024, 64, 2, 704, 2816, 128
NC, NS, L = 2, 16, 16
NW = NC * NS
BT = 256
NB = T // BT
TPW = T // NW
RB = 32
CH = 16
XROWS = (E + 1) * C


def _sigmoid(v):
    return 1.0 / (1.0 + jnp.exp(-v))


def _router_rank_body(x_ref, gw_ref, tril_ref, sd1_ref, sd2_ref,
                      sc1_ref, sc2_ref, w1_ref, w2_ref, carry_ref):
    b = pl.program_id(0)

    @pl.when(b == 0)
    def _():
        carry_ref[...] = jnp.zeros_like(carry_ref)

    x = x_ref[...]
    logits = jnp.dot(x, gw_ref[...], preferred_element_type=jnp.float32)
    m = jnp.max(logits, axis=-1, keepdims=True)
    ex = jnp.exp(logits - m)
    probs = ex / jnp.sum(ex, axis=-1, keepdims=True)

    iota = lax.broadcasted_iota(jnp.int32, probs.shape, 1)
    m1 = jnp.max(probs, axis=-1, keepdims=True)
    i1 = jnp.min(jnp.where(probs == m1, iota, E), axis=-1, keepdims=True)
    probs2 = jnp.where(iota == i1, -jnp.inf, probs)
    m2 = jnp.max(probs2, axis=-1, keepdims=True)
    i2 = jnp.min(jnp.where(probs2 == m2, iota, E), axis=-1, keepdims=True)

    oh1 = (i1 == iota).astype(jnp.float32)
    oh2 = (i2 == iota).astype(jnp.float32)
    s = oh1 + oh2
    csum_ex = (jnp.dot(tril_ref[...], s, preferred_element_type=jnp.float32)
               + carry_ref[0:1, :])
    carry_ref[0:1, :] = carry_ref[0:1, :] + jnp.sum(s, axis=0, keepdims=True)

    r1 = jnp.sum(oh1 * csum_ex, axis=-1, keepdims=True).astype(jnp.int32)
    r2 = jnp.sum(oh2 * csum_ex, axis=-1, keepdims=True).astype(jnp.int32)

    tt = lax.broadcasted_iota(jnp.int32, r1.shape, 0)
    sink = E * C + (b * BT + tt) // TPW
    k1 = r1 < C
    k2 = r2 < C
    slot1 = i1 * C + r1
    slot2 = i2 * C + r2
    sd1_ref[0, 0, :] = jnp.where(k1, slot1, sink)[:, 0]
    sd2_ref[0, 0, :] = jnp.where(k2, slot2, sink)[:, 0]
    sc1_ref[0, 0, :] = jnp.where(k1, slot1, 0)[:, 0]
    sc2_ref[0, 0, :] = jnp.where(k2, slot2, 0)[:, 0]
    w1_ref[0, :, :] = jnp.broadcast_to(jnp.where(k1, m1, 0.0), (BT, L))
    w2_ref[0, :, :] = jnp.broadcast_to(jnp.where(k2, m2, 0.0), (BT, L))


def _router_rank(x, gate_w, tril):
    outs = pl.pallas_call(
        _router_rank_body,
        grid=(NB,),
        in_specs=[pl.BlockSpec((BT, D), lambda b: (b, 0)),
                  pl.BlockSpec((D, E), lambda b: (0, 0)),
                  pl.BlockSpec((BT, BT), lambda b: (0, 0))],
        out_specs=[pl.BlockSpec((1, 1, BT), lambda b: (b, 0, 0))] * 4
        + [pl.BlockSpec((1, BT, L), lambda b: (b, 0, 0))] * 2,
        out_shape=[jax.ShapeDtypeStruct((NB, 1, BT), jnp.int32),
                   jax.ShapeDtypeStruct((NB, 1, BT), jnp.int32),
                   jax.ShapeDtypeStruct((NB, 1, BT), jnp.int32),
                   jax.ShapeDtypeStruct((NB, 1, BT), jnp.int32),
                   jax.ShapeDtypeStruct((NB, BT, L), jnp.float32),
                   jax.ShapeDtypeStruct((NB, BT, L), jnp.float32)],
        scratch_shapes=[pltpu.VMEM((8, E), jnp.float32)],
    )(x, gate_w, tril)
    return (outs[0].reshape(T), outs[1].reshape(T), outs[2].reshape(T),
            outs[3].reshape(T), outs[4].reshape(T, L), outs[5].reshape(T, L))


def _dispatch(sd1, sd2, x):
    mesh = plsc.VectorSubcoreMesh(core_axis_name="c", subcore_axis_name="s")

    @functools.partial(
        pl.kernel,
        mesh=mesh,
        out_type=jax.ShapeDtypeStruct((XROWS, D), jnp.float32),
        scratch_types=(
            pltpu.VMEM((RB, D), jnp.float32),
            pltpu.VMEM((RB,), jnp.int32),
            pltpu.VMEM((RB,), jnp.int32),
            pltpu.SemaphoreType.DMA,
        ),
    )
    def k(sd1_hbm, sd2_hbm, x_hbm, xe_hbm, rows_v, i1_v, i2_v, sem):
        wid = lax.axis_index("s") * NC + lax.axis_index("c")
        for ch in range(TPW // RB):
            base = wid * TPW + ch * RB
            pltpu.sync_copy(x_hbm.at[pl.ds(base, RB)], rows_v)
            pltpu.sync_copy(sd1_hbm.at[pl.ds(base, RB)], i1_v)
            pltpu.sync_copy(sd2_hbm.at[pl.ds(base, RB)], i2_v)
            c1 = pltpu.async_copy(rows_v, xe_hbm.at[i1_v], sem)
            c2 = pltpu.async_copy(rows_v, xe_hbm.at[i2_v], sem)
            c1.wait()
            c2.wait()

    return k(sd1, sd2, x)


D2 = D // 2


def _experts_body(x_ref, w1a_ref, w1b_ref, w3a_ref, w3b_ref,
                  w2a_ref, w2b_ref, o_ref):
    x = x_ref[...]
    xl = x[:, :D2]
    xr = x[:, D2:]
    g = (jnp.dot(xl, w1a_ref[0], preferred_element_type=jnp.float32)
         + jnp.dot(xr, w1b_ref[0], preferred_element_type=jnp.float32))
    u = (jnp.dot(xl, w3a_ref[0], preferred_element_type=jnp.float32)
         + jnp.dot(xr, w3b_ref[0], preferred_element_type=jnp.float32))
    h = g * _sigmoid(g) * u
    o_ref[:, :D2] = jnp.dot(h, w2a_ref[0], preferred_element_type=jnp.float32)
    o_ref[:, D2:] = jnp.dot(h, w2b_ref[0], preferred_element_type=jnp.float32)


def _experts(xe, w1, w3, w2):
    return pl.pallas_call(
        _experts_body,
        grid=(E,),
        in_specs=[pl.BlockSpec((C, D), lambda e: (e, 0)),
                  pl.BlockSpec((1, D2, FF), lambda e: (e, 0, 0)),
                  pl.BlockSpec((1, D2, FF), lambda e: (e, 1, 0)),
                  pl.BlockSpec((1, D2, FF), lambda e: (e, 0, 0)),
                  pl.BlockSpec((1, D2, FF), lambda e: (e, 1, 0)),
                  pl.BlockSpec((1, FF, D2), lambda e: (e, 0, 0)),
                  pl.BlockSpec((1, FF, D2), lambda e: (e, 0, 1))],
        out_specs=pl.BlockSpec((C, D), lambda e: (e, 0)),
        out_shape=jax.ShapeDtypeStruct((E * C, D), jnp.float32),
        compiler_params=pltpu.CompilerParams(
            dimension_semantics=("parallel",)),
    )(xe, w1, w1, w3, w3, w2, w2)


def _shared_body(x_ref, w1_ref, w3_ref, w2_ref, gw_ref, o_ref):
    x = x_ref[...]
    s1 = jnp.dot(x, w1_ref[...], preferred_element_type=jnp.float32)
    s3 = jnp.dot(x, w3_ref[...], preferred_element_type=jnp.float32)
    h = s1 * _sigmoid(s1) * s3
    so = jnp.dot(h, w2_ref[...], preferred_element_type=jnp.float32)
    gl = jnp.dot(x, gw_ref[...], preferred_element_type=jnp.float32)
    o_ref[...] = so * _sigmoid(gl)


def _shared(x, sw1, sw3, sw2, sgate_w):
    return pl.pallas_call(
        _shared_body,
        grid=(T // BT,),
        in_specs=[pl.BlockSpec((BT, D), lambda i: (i, 0)),
                  pl.BlockSpec((D, FFS), lambda i: (0, 0)),
                  pl.BlockSpec((D, FFS), lambda i: (0, 0)),
                  pl.BlockSpec((FFS, D), lambda i: (0, 0)),
                  pl.BlockSpec((D, 1), lambda i: (0, 0))],
        out_specs=pl.BlockSpec((BT, D), lambda i: (i, 0)),
        out_shape=jax.ShapeDtypeStruct((T, D), jnp.float32),
        compiler_params=pltpu.CompilerParams(
            dimension_semantics=("parallel",)),
    )(x, sw1, sw3, sw2, sgate_w)


def _gather2(oe, sc1, sc2):
    mesh = plsc.VectorSubcoreMesh(core_axis_name="c", subcore_axis_name="s")

    @functools.partial(
        pl.kernel,
        mesh=mesh,
        out_type=(jax.ShapeDtypeStruct((T, D), jnp.float32),
                  jax.ShapeDtypeStruct((T, D), jnp.float32)),
        scratch_types=(
            pltpu.VMEM((CH,), jnp.int32),
            pltpu.VMEM((CH,), jnp.int32),
            pltpu.VMEM((CH, D), jnp.float32),
            pltpu.VMEM((CH, D), jnp.float32),
            pltpu.SemaphoreType.DMA,
        ),
    )
    def k(oe_hbm, sc1_hbm, sc2_hbm, ga_hbm, gb_hbm,
          s1_v, s2_v, ra_v, rb_v, sem):
        wid = lax.axis_index("s") * NC + lax.axis_index("c")
        for ch in range(TPW // CH):
            tb = wid * TPW + ch * CH
            pltpu.sync_copy(sc1_hbm.at[pl.ds(tb, CH)], s1_v)
            pltpu.sync_copy(sc2_hbm.at[pl.ds(tb, CH)], s2_v)
            g1 = pltpu.async_copy(oe_hbm.at[s1_v], ra_v, sem)
            g2 = pltpu.async_copy(oe_hbm.at[s2_v], rb_v, sem)
            g1.wait()
            g2.wait()
            pltpu.sync_copy(ra_v, ga_hbm.at[pl.ds(tb, CH)])
            pltpu.sync_copy(rb_v, gb_hbm.at[pl.ds(tb, CH)])

    return k(oe, sc1, sc2)


def _weighted_add_body(ga_ref, gb_ref, w1_ref, w2_ref, so_ref, o_ref):
    wa = w1_ref[:, 0:1]
    wb = w2_ref[:, 0:1]
    o_ref[...] = (jnp.where(wa != 0.0, ga_ref[...] * wa, 0.0)
                  + jnp.where(wb != 0.0, gb_ref[...] * wb, 0.0)
                  + so_ref[...])


def _weighted_add(ga, gb, w1p, w2p, so):
    return pl.pallas_call(
        _weighted_add_body,
        grid=(T // BT,),
        in_specs=[pl.BlockSpec((BT, D), lambda i: (i, 0)),
                  pl.BlockSpec((BT, D), lambda i: (i, 0)),
                  pl.BlockSpec((BT, L), lambda i: (i, 0)),
                  pl.BlockSpec((BT, L), lambda i: (i, 0)),
                  pl.BlockSpec((BT, D), lambda i: (i, 0))],
        out_specs=pl.BlockSpec((BT, D), lambda i: (i, 0)),
        out_shape=jax.ShapeDtypeStruct((T, D), jnp.float32),
        compiler_params=pltpu.CompilerParams(
            dimension_semantics=("parallel",)),
    )(ga, gb, w1p, w2p, so)


def kernel(hidden_states, gate_w, w1, w3, w2, sw1, sw3, sw2, sgate_w):
    tril = jnp.tril(jnp.ones((BT, BT), jnp.float32), -1)
    sd1, sd2, sc1, sc2, w1p, w2p = _router_rank(hidden_states, gate_w, tril)
    xe = _dispatch(sd1, sd2, hidden_states)
    oe = _experts(xe, w1, w3, w2)
    so = _shared(hidden_states, sw1, sw3, sw2, sgate_w)
    ga, gb = _gather2(oe, sc1, sc2)
    return _weighted_add(ga, gb, w1p, w2p, so)
